# baseline (speedup 1.0000x reference)
import jax
import jax.numpy as jnp
from jax.experimental import pallas as pl

_B = 64
_N = 100
_D = 256


def _tc_body(emb_ref, zt_ref, pet_ref):
    zt_ref[...] = jnp.zeros(zt_ref.shape, zt_ref.dtype)
    pet_ref[...] = jnp.broadcast_to(emb_ref[...], pet_ref.shape)


def kernel(x, emb_table):
    nb = 100
    out_t = jax.ShapeDtypeStruct((_N, _B, _D), jnp.float32)
    zt, pet = pl.pallas_call(
        _tc_body,
        grid=(_N // nb,),
        in_specs=[pl.BlockSpec((nb, 1, _D), lambda i: (i, 0, 0))],
        out_specs=[
            pl.BlockSpec((nb, _B, _D), lambda i: (i, 0, 0)),
            pl.BlockSpec((nb, _B, _D), lambda i: (i, 0, 0)),
        ],
        out_shape=[out_t, out_t],
    )(emb_table.reshape(_N, 1, _D))
    return (jnp.transpose(zt, (1, 0, 2)), jnp.transpose(pet, (1, 0, 2)))


# speedup vs baseline: 1.0191x; 1.0191x over previous
import jax
import jax.numpy as jnp
from jax.experimental import pallas as pl

_B = 64
_N = 100
_D = 256


def _tc_body(emb_ref, zt_ref, pet_ref):
    zt_ref[...] = jnp.zeros(zt_ref.shape, zt_ref.dtype)
    pet_ref[...] = jnp.broadcast_to(emb_ref[...], pet_ref.shape)


def kernel(x, emb_table):
    nb = 25
    out_t = jax.ShapeDtypeStruct((_N, _B, _D), jnp.float32)
    zt, pet = pl.pallas_call(
        _tc_body,
        grid=(_N // nb,),
        in_specs=[pl.BlockSpec((nb, 1, _D), lambda i: (i, 0, 0))],
        out_specs=[
            pl.BlockSpec((nb, _B, _D), lambda i: (i, 0, 0)),
            pl.BlockSpec((nb, _B, _D), lambda i: (i, 0, 0)),
        ],
        out_shape=[out_t, out_t],
    )(emb_table.reshape(_N, 1, _D))
    return (jnp.transpose(zt, (1, 0, 2)), jnp.transpose(pet, (1, 0, 2)))


# speedup vs baseline: 1.0319x; 1.0125x over previous
import jax
import jax.numpy as jnp
from jax.experimental import pallas as pl


def _tc_body(emb_ref, zt_ref, pet_ref):
    zt_ref[...] = jnp.zeros(zt_ref.shape, zt_ref.dtype)
    pet_ref[...] = jnp.broadcast_to(emb_ref[...], pet_ref.shape)


def kernel(x, emb_table):
    b = x.shape[0]
    n, d = emb_table.shape
    nb = 50 if n % 50 == 0 else n
    out_t = jax.ShapeDtypeStruct((n, b, d), jnp.float32)
    zt, pet = pl.pallas_call(
        _tc_body,
        grid=(n // nb,),
        in_specs=[pl.BlockSpec((nb, 1, d), lambda i: (i, 0, 0))],
        out_specs=[
            pl.BlockSpec((nb, b, d), lambda i: (i, 0, 0)),
            pl.BlockSpec((nb, b, d), lambda i: (i, 0, 0)),
        ],
        out_shape=[out_t, out_t],
    )(emb_table.reshape(n, 1, d))
    return (jnp.transpose(zt, (1, 0, 2)), jnp.transpose(pet, (1, 0, 2)))
